# baseline (speedup 1.0000x reference)
import functools
import jax
import jax.numpy as jnp
from jax import lax
from jax.experimental import pallas as pl
from jax.experimental.pallas import tpu as pltpu
from jax.experimental.pallas import tpu_sc as plsc

N, C, L = 16, 2048, 512
G = 32
HALF = C // 2
MAXCH = HALF // G
IDXLEN = HALF + G
NV = L // 16


def _make_sc_kernel():
    mesh = plsc.VectorSubcoreMesh(core_axis_name="c", subcore_axis_name="s")

    @functools.partial(
        pl.kernel,
        mesh=mesh,
        out_type=jax.ShapeDtypeStruct((32, 16), jnp.float32),
        compiler_params=pltpu.CompilerParams(needs_layout_passes=False),
        scratch_types=[
            pltpu.VMEM((C,), jnp.int32),
            pltpu.VMEM((IDXLEN,), jnp.int32),
            pltpu.VMEM((G, L), jnp.float32),
            pltpu.VMEM((G, L), jnp.float32),
            pltpu.VMEM((G, L), jnp.float32),
            pltpu.VMEM((G, L), jnp.float32),
            pltpu.VMEM((16,), jnp.float32),
            pltpu.SemaphoreType.DMA,
            pltpu.SemaphoreType.DMA,
        ],
    )
    def sc_kern(s_hbm, t_hbm, tgt_hbm, out_hbm,
                tgt_v, idx_v, s0, t0, s1, t1, acc_v, sem0, sem1):
        wid = lax.axis_index("s") * 2 + lax.axis_index("c")
        inst = wid // 2
        half = wid % 2
        row_base = inst * C + half * HALF

        pltpu.sync_copy(tgt_hbm.at[pl.ds(inst * C, C)], tgt_v)

        def npos_body(i, s):
            m = tgt_v[pl.ds(i * 16, 16)] > 0
            return s + plsc.all_reduce_population_count(m)
        npos_vec = lax.fori_loop(0, C // 16, npos_body,
                                 jnp.zeros((16,), jnp.int32))
        w_vec = jnp.where(npos_vec > 1,
                          1.0 / (npos_vec.astype(jnp.float32) * float(L)),
                          jnp.zeros((16,), jnp.float32))

        zero16 = jnp.zeros((16,), jnp.int32)
        def zero_body(i, c):
            idx_v[pl.ds(i * 16, 16)] = zero16
            return c
        lax.fori_loop(0, IDXLEN // 16, zero_body, 0)

        iota = lax.iota(jnp.int32, 16)
        def comp_body(j, cnt):
            v = tgt_v[pl.ds(half * HALF + j * 16, 16)]
            m = v > 0
            vals = (row_base + j * 16) + iota
            plsc.store_compressed(idx_v.at[pl.ds(cnt, 16)], vals, mask=m)
            c = plsc.all_reduce_population_count(m)
            return cnt + c[0]
        cnt = lax.fori_loop(0, HALF // 16, comp_body, jnp.int32(0))

        nchunks = (cnt + (G - 1)) >> 5
        npairs = (nchunks + 1) >> 1

        acc_v[...] = jnp.zeros((16,), jnp.float32)

        def start_chunk(chunk, sb, tb, sem):
            off = pl.multiple_of(chunk * G, G)
            idxs = idx_v.at[pl.ds(off, G)]
            pltpu.async_copy(s_hbm.at[idxs], sb, sem)
            pltpu.async_copy(t_hbm.at[idxs], tb, sem)

        def wait_chunk(sb, tb, sem):
            idxs = idx_v.at[pl.ds(0, G)]
            pltpu.make_async_copy(s_hbm.at[idxs], sb, sem).wait()
            pltpu.make_async_copy(t_hbm.at[idxs], tb, sem).wait()

        def compute_chunk(chunk, sb, tb):
            valid = jnp.minimum(cnt - chunk * G, G)
            def row_body(r, racc):
                def v_body(k, vacc):
                    s = sb[r, pl.ds(k * 16, 16)]
                    t = tb[r, pl.ds(k * 16, 16)]
                    a = jnp.abs(s - t)
                    mn = jnp.minimum(a, 1.0)
                    return vacc + mn * (a - 0.5 * mn)
                return lax.fori_loop(0, NV, v_body, racc)
            local = lax.fori_loop(0, valid, row_body,
                                  jnp.zeros((16,), jnp.float32))
            acc_v[...] = acc_v[...] + local

        @pl.when(nchunks > 0)
        def _():
            start_chunk(0, s0, t0, sem0)

        @pl.when(nchunks > 1)
        def _():
            start_chunk(1, s1, t1, sem1)

        def pair_body(p, c):
            a = 2 * p
            wait_chunk(s0, t0, sem0)
            compute_chunk(a, s0, t0)

            @pl.when(a + 2 < nchunks)
            def _():
                start_chunk(a + 2, s0, t0, sem0)

            b = a + 1

            @pl.when(b < nchunks)
            def _():
                wait_chunk(s1, t1, sem1)
                compute_chunk(b, s1, t1)

                @pl.when(b + 2 < nchunks)
                def _():
                    start_chunk(b + 2, s1, t1, sem1)

            return c
        lax.fori_loop(0, npairs, pair_body, 0)

        acc_v[...] = acc_v[...] * w_vec
        pltpu.sync_copy(acc_v, out_hbm.at[wid])

    return sc_kern


_sc_kernel = _make_sc_kernel()


def kernel(le_student, le_teacher, targets):
    s = le_student.reshape(N * C, L)
    t = le_teacher.reshape(N * C, L)
    tgt = targets.reshape(N * C)
    out = _sc_kernel(s, t, tgt)
    return jnp.sum(out)


# speedup vs baseline: 1.4036x; 1.4036x over previous
import functools
import jax
import jax.numpy as jnp
from jax import lax
from jax.experimental import pallas as pl
from jax.experimental.pallas import tpu as pltpu
from jax.experimental.pallas import tpu_sc as plsc

N, C, L = 16, 2048, 512
G = 32
HALF = C // 2
MAXCH = HALF // G
IDXLEN = HALF + G
NV = L // 16


def _make_sc_kernel():
    mesh = plsc.VectorSubcoreMesh(core_axis_name="c", subcore_axis_name="s")

    @functools.partial(
        pl.kernel,
        mesh=mesh,
        out_type=jax.ShapeDtypeStruct((32, 16), jnp.float32),
        compiler_params=pltpu.CompilerParams(needs_layout_passes=False),
        scratch_types=[
            pltpu.VMEM((C,), jnp.int32),
            pltpu.VMEM((IDXLEN,), jnp.int32),
            pltpu.VMEM((G, L), jnp.float32),
            pltpu.VMEM((G, L), jnp.float32),
            pltpu.VMEM((G, L), jnp.float32),
            pltpu.VMEM((G, L), jnp.float32),
            pltpu.VMEM((16,), jnp.float32),
            pltpu.SemaphoreType.DMA,
            pltpu.SemaphoreType.DMA,
        ],
    )
    def sc_kern(s_hbm, t_hbm, tgt_hbm, out_hbm,
                tgt_v, idx_v, s0, t0, s1, t1, acc_v, sem0, sem1):
        wid = lax.axis_index("s") * 2 + lax.axis_index("c")
        inst = wid // 2
        half = wid % 2
        row_base = inst * C + half * HALF

        pltpu.sync_copy(tgt_hbm.at[pl.ds(inst * C, C)], tgt_v)

        def npos_body(i, s):
            m = tgt_v[pl.ds(i * 16, 16)] > 0
            return s + plsc.all_reduce_population_count(m)
        npos_vec = lax.fori_loop(0, C // 16, npos_body,
                                 jnp.zeros((16,), jnp.int32))
        w_vec = jnp.where(npos_vec > 1,
                          1.0 / (npos_vec.astype(jnp.float32) * float(L)),
                          jnp.zeros((16,), jnp.float32))

        zero16 = jnp.zeros((16,), jnp.int32)
        def zero_body(i, c):
            idx_v[pl.ds(i * 16, 16)] = zero16
            return c
        lax.fori_loop(0, IDXLEN // 16, zero_body, 0)

        iota = lax.iota(jnp.int32, 16)
        def comp_body(j, cnt):
            v = tgt_v[pl.ds(half * HALF + j * 16, 16)]
            m = v > 0
            vals = (row_base + j * 16) + iota
            plsc.store_compressed(idx_v.at[pl.ds(cnt, 16)], vals, mask=m)
            c = plsc.all_reduce_population_count(m)
            return cnt + c[0]
        cnt = lax.fori_loop(0, HALF // 16, comp_body, jnp.int32(0))

        nchunks = (cnt + (G - 1)) >> 5
        npairs = (nchunks + 1) >> 1

        acc_v[...] = jnp.zeros((16,), jnp.float32)

        def start_chunk(chunk, sb, tb, sem):
            off = pl.multiple_of(chunk * G, G)
            idxs = idx_v.at[pl.ds(off, G)]
            pltpu.async_copy(s_hbm.at[idxs], sb, sem)
            pltpu.async_copy(t_hbm.at[idxs], tb, sem)

        def wait_chunk(sb, tb, sem):
            idxs = idx_v.at[pl.ds(0, G)]
            pltpu.make_async_copy(s_hbm.at[idxs], sb, sem).wait()
            pltpu.make_async_copy(t_hbm.at[idxs], tb, sem).wait()

        def compute_chunk(chunk, sb, tb):
            valid = jnp.minimum(cnt - chunk * G, G)
            def row_body(r, racc):
                acc = racc
                for k in range(NV):
                    s = sb[r, pl.ds(k * 16, 16)]
                    t = tb[r, pl.ds(k * 16, 16)]
                    a = jnp.abs(s - t)
                    mn = jnp.minimum(a, 1.0)
                    acc = acc + mn * (a - 0.5 * mn)
                return acc
            local = lax.fori_loop(0, valid, row_body,
                                  jnp.zeros((16,), jnp.float32))
            acc_v[...] = acc_v[...] + local

        @pl.when(nchunks > 0)
        def _():
            start_chunk(0, s0, t0, sem0)

        @pl.when(nchunks > 1)
        def _():
            start_chunk(1, s1, t1, sem1)

        def pair_body(p, c):
            a = 2 * p
            wait_chunk(s0, t0, sem0)
            compute_chunk(a, s0, t0)

            @pl.when(a + 2 < nchunks)
            def _():
                start_chunk(a + 2, s0, t0, sem0)

            b = a + 1

            @pl.when(b < nchunks)
            def _():
                wait_chunk(s1, t1, sem1)
                compute_chunk(b, s1, t1)

                @pl.when(b + 2 < nchunks)
                def _():
                    start_chunk(b + 2, s1, t1, sem1)

            return c
        lax.fori_loop(0, npairs, pair_body, 0)

        acc_v[...] = acc_v[...] * w_vec
        pltpu.sync_copy(acc_v, out_hbm.at[wid])

    return sc_kern


_sc_kernel = _make_sc_kernel()


def kernel(le_student, le_teacher, targets):
    s = le_student.reshape(N * C, L)
    t = le_teacher.reshape(N * C, L)
    tgt = targets.reshape(N * C)
    out = _sc_kernel(s, t, tgt)
    return jnp.sum(out)


# speedup vs baseline: 1.5341x; 1.0930x over previous
import functools
import jax
import jax.numpy as jnp
from jax import lax
from jax.experimental import pallas as pl
from jax.experimental.pallas import tpu as pltpu
from jax.experimental.pallas import tpu_sc as plsc

N, C, L = 16, 2048, 512
G = 32
HALF = C // 2
MAXCH = HALF // G
IDXLEN = HALF + G
NV = L // 16


def _make_sc_kernel():
    mesh = plsc.VectorSubcoreMesh(core_axis_name="c", subcore_axis_name="s")

    @functools.partial(
        pl.kernel,
        mesh=mesh,
        out_type=jax.ShapeDtypeStruct((32, 16), jnp.float32),
        compiler_params=pltpu.CompilerParams(needs_layout_passes=False),
        scratch_types=[
            pltpu.VMEM((C,), jnp.int32),
            pltpu.VMEM((IDXLEN,), jnp.int32),
            pltpu.VMEM((G, L), jnp.float32),
            pltpu.VMEM((G, L), jnp.float32),
            pltpu.VMEM((G, L), jnp.float32),
            pltpu.VMEM((G, L), jnp.float32),
            pltpu.VMEM((G, L), jnp.float32),
            pltpu.VMEM((G, L), jnp.float32),
            pltpu.VMEM((16,), jnp.float32),
            pltpu.SemaphoreType.DMA,
            pltpu.SemaphoreType.DMA,
            pltpu.SemaphoreType.DMA,
        ],
    )
    def sc_kern(s_hbm, t_hbm, tgt_hbm, out_hbm,
                tgt_v, idx_v, s0, t0, s1, t1, s2, t2, acc_v,
                sem0, sem1, sem2):
        wid = lax.axis_index("s") * 2 + lax.axis_index("c")
        inst = wid // 2
        half = wid % 2
        row_base = inst * C + half * HALF

        pltpu.sync_copy(tgt_hbm.at[pl.ds(inst * C, C)], tgt_v)

        def npos_body(i, s):
            m = tgt_v[pl.ds(i * 16, 16)] > 0
            return s + plsc.all_reduce_population_count(m)
        npos_vec = lax.fori_loop(0, C // 16, npos_body,
                                 jnp.zeros((16,), jnp.int32))
        w_vec = jnp.where(npos_vec > 1,
                          1.0 / (npos_vec.astype(jnp.float32) * float(L)),
                          jnp.zeros((16,), jnp.float32))

        zero16 = jnp.zeros((16,), jnp.int32)
        def zero_body(i, c):
            idx_v[pl.ds(i * 16, 16)] = zero16
            return c
        lax.fori_loop(0, IDXLEN // 16, zero_body, 0)

        iota = lax.iota(jnp.int32, 16)
        def comp_body(j, cnt):
            v = tgt_v[pl.ds(half * HALF + j * 16, 16)]
            m = v > 0
            vals = (row_base + j * 16) + iota
            plsc.store_compressed(idx_v.at[pl.ds(cnt, 16)], vals, mask=m)
            c = plsc.all_reduce_population_count(m)
            return cnt + c[0]
        cnt = lax.fori_loop(0, HALF // 16, comp_body, jnp.int32(0))

        nchunks = (cnt + (G - 1)) >> 5
        ntriples = (nchunks + 2) // 3

        acc_v[...] = jnp.zeros((16,), jnp.float32)

        def start_chunk(chunk, sb, tb, sem):
            off = pl.multiple_of(chunk * G, G)
            idxs = idx_v.at[pl.ds(off, G)]
            pltpu.async_copy(s_hbm.at[idxs], sb, sem)
            pltpu.async_copy(t_hbm.at[idxs], tb, sem)

        def wait_chunk(sb, tb, sem):
            idxs = idx_v.at[pl.ds(0, G)]
            pltpu.make_async_copy(s_hbm.at[idxs], sb, sem).wait()
            pltpu.make_async_copy(t_hbm.at[idxs], tb, sem).wait()

        def compute_chunk(chunk, sb, tb):
            valid = jnp.minimum(cnt - chunk * G, G)
            def row_body(r, racc):
                acc = racc
                for k in range(NV):
                    s = sb[r, pl.ds(k * 16, 16)]
                    t = tb[r, pl.ds(k * 16, 16)]
                    a = jnp.abs(s - t)
                    mn = jnp.minimum(a, 1.0)
                    acc = acc + mn * (a - 0.5 * mn)
                return acc
            local = lax.fori_loop(0, valid, row_body,
                                  jnp.zeros((16,), jnp.float32))
            acc_v[...] = acc_v[...] + local

        bufs = ((s0, t0, sem0), (s1, t1, sem1), (s2, t2, sem2))
        RING = len(bufs)

        for b in range(RING):
            sb, tb, sem = bufs[b]

            @pl.when(b < nchunks)
            def _(b=b, sb=sb, tb=tb, sem=sem):
                start_chunk(b, sb, tb, sem)

        def ring_body(p, c):
            base = RING * p
            for b in range(RING):
                sb, tb, sem = bufs[b]
                chunk = base + b

                @pl.when(chunk < nchunks)
                def _(chunk=chunk, sb=sb, tb=tb, sem=sem):
                    wait_chunk(sb, tb, sem)
                    compute_chunk(chunk, sb, tb)

                    @pl.when(chunk + RING < nchunks)
                    def _():
                        start_chunk(chunk + RING, sb, tb, sem)

            return c
        lax.fori_loop(0, ntriples, ring_body, 0)

        acc_v[...] = acc_v[...] * w_vec
        pltpu.sync_copy(acc_v, out_hbm.at[wid])

    return sc_kern


_sc_kernel = _make_sc_kernel()


def kernel(le_student, le_teacher, targets):
    s = le_student.reshape(N * C, L)
    t = le_teacher.reshape(N * C, L)
    tgt = targets.reshape(N * C)
    out = _sc_kernel(s, t, tgt)
    return jnp.sum(out)
